# Initial kernel scaffold; baseline (speedup 1.0000x reference)
import jax
import jax.numpy as jnp
from jax.experimental import pallas as pl


def kernel(node_feature, edge_index, edge_type, node_type, params):
    raise NotImplementedError("write your pallas kernel here")


# speedup vs baseline: 4.3129x; 4.3129x over previous
import functools

import jax
import jax.numpy as jnp
from jax import lax
from jax.experimental import pallas as pl
from jax.experimental.pallas import tpu as pltpu
from jax.experimental.pallas import tpu_sc as plsc

_N = 10000
_E = 320000
_D = 128
_H = 128
_NT = 2
_ET = 3

_NC = 2
_NS = 16
_L = 16
_DH = _D // 2

_NH = _N // _NC
_ACC_ROWS = _ET * _NH
_TRASH = _ACC_ROWS

_CHUNK = 128
_ACC_PAD = -(-(_ACC_ROWS + 1) // _CHUNK) * _CHUNK

_NCHUNK = 2 * (-(-_E // (_NS * _CHUNK * 2)))
_NPAIR = _NCHUNK // 2
_EPT = _NCHUNK * _CHUNK
_EPAD = _EPT * _NS

_ZCHUNKS = _ACC_PAD // _CHUNK
_ZCOPIES = -(-_ZCHUNKS // _NS)
_WB = 320


def _edge_mlp_body(x_ref, w1_ref, b1_ref, w2_ref, b2_ref, lo_ref, hi_ref):
    x = x_ref[...]
    h = jnp.dot(x, w1_ref[0], preferred_element_type=jnp.float32) + b1_ref[0, 0]
    h = jnp.maximum(h, 0.0)
    o = jnp.dot(h, w2_ref[0], preferred_element_type=jnp.float32) + b2_ref[0, 0]
    o = jnp.maximum(o, 0.0)
    lo_ref[0] = o[:, :_DH]
    hi_ref[0] = o[:, _DH:]


def _edge_tables(x, w1s, b1s, w2s, b2s):
    bn = 2000
    nb = _N // bn
    return pl.pallas_call(
        _edge_mlp_body,
        grid=(_ET, nb),
        in_specs=[
            pl.BlockSpec((bn, _D), lambda i, j: (j, 0)),
            pl.BlockSpec((1, _D, _H), lambda i, j: (i, 0, 0)),
            pl.BlockSpec((1, 1, _H), lambda i, j: (i, 0, 0)),
            pl.BlockSpec((1, _H, _D), lambda i, j: (i, 0, 0)),
            pl.BlockSpec((1, 1, _D), lambda i, j: (i, 0, 0)),
        ],
        out_specs=[
            pl.BlockSpec((1, bn, _DH), lambda i, j: (i, j, 0)),
            pl.BlockSpec((1, bn, _DH), lambda i, j: (i, j, 0)),
        ],
        out_shape=[
            jax.ShapeDtypeStruct((_ET, _N, _DH), jnp.float32),
            jax.ShapeDtypeStruct((_ET, _N, _DH), jnp.float32),
        ],
    )(x, w1s, b1s, w2s, b2s)


def _idx_body(src_ref, dst_ref, typ_ref, gidx_ref, sidx_ref):
    src = src_ref[...]
    dst = dst_ref[...]
    typ = typ_ref[...]
    gidx_ref[...] = typ * _N + src
    rel0 = typ * _NH + dst
    sidx_ref[0] = jnp.where((dst >= 0) & (dst < _NH), rel0, _TRASH)
    sidx_ref[1] = jnp.where(dst >= _NH, rel0 - _NH, _TRASH)


def _edge_indices(src, dst, typ):
    rows = _EPAD // _CHUNK
    src2 = src.reshape(rows, _CHUNK)
    dst2 = dst.reshape(rows, _CHUNK)
    typ2 = typ.reshape(rows, _CHUNK)
    gidx, sidx = pl.pallas_call(
        _idx_body,
        out_shape=[
            jax.ShapeDtypeStruct((rows, _CHUNK), jnp.int32),
            jax.ShapeDtypeStruct((2, rows, _CHUNK), jnp.int32),
        ],
    )(src2, dst2, typ2)
    return (gidx.reshape(_NS, _NCHUNK, _CHUNK),
            sidx.reshape(2, _NS, _NCHUNK, _CHUNK))


def _agg_body(mlo_hbm, mhi_hbm, gidx_hbm, sidx_hbm, out_hbm,
              gidx_v, sidx_v, rows_v, acc_sh, sem0, sem1):
    cid = lax.axis_index("c")
    sid = lax.axis_index("s")
    base = cid * _NH

    pltpu.sync_copy(gidx_hbm.at[sid], gidx_v)
    pltpu.sync_copy(sidx_hbm.at[cid, sid], sidx_v)

    for p in range(2):
        m_hbm = mlo_hbm if p == 0 else mhi_hbm

        def start(c, buf, sem, m_hbm=m_hbm):
            pltpu.async_copy(m_hbm.at[gidx_v.at[c]], rows_v.at[buf], sem)

        def wait(buf, sem, m_hbm=m_hbm):
            pltpu.make_async_copy(m_hbm.at[gidx_v.at[0]], rows_v.at[buf], sem).wait()

        zero = jnp.zeros((_L,), jnp.float32)

        def zrow(j, _):
            for k in range(_DH // _L):
                rows_v[0, j, pl.ds(k * _L, _L)] = zero
            return 0

        lax.fori_loop(0, _CHUNK, zrow, 0)

        def zcopy(j, _):
            idx = jnp.minimum(sid * _ZCOPIES + j, _ZCHUNKS - 1)
            zstart = pl.multiple_of(idx * _CHUNK, _CHUNK)
            pltpu.sync_copy(rows_v.at[0], acc_sh.at[pl.ds(zstart, _CHUNK)])
            return 0

        lax.fori_loop(0, _ZCOPIES, zcopy, 0)
        plsc.subcore_barrier()

        def pair(i, _):
            c0 = 2 * i
            start(c0 + 1, 1, sem1)
            wait(0, sem0)
            pltpu.sync_copy(rows_v.at[0], acc_sh.at[sidx_v.at[c0]], add=True)

            @pl.when(i + 1 < _NPAIR)
            def _():
                start(c0 + 2, 0, sem0)

            wait(1, sem1)
            pltpu.sync_copy(rows_v.at[1], acc_sh.at[sidx_v.at[c0 + 1]], add=True)
            return 0

        start(0, 0, sem0)
        lax.fori_loop(0, _NPAIR, pair, 0)
        plsc.subcore_barrier()

        wstart = pl.multiple_of(jnp.minimum(sid * _WB, _NH - _WB), 8)
        for i in range(_ET):
            pltpu.sync_copy(acc_sh.at[pl.ds(i * _NH + wstart, _WB)],
                            out_hbm.at[p, i, pl.ds(base + wstart, _WB)])
        plsc.subcore_barrier()


def _aggregate(m_lo, m_hi, gidx, sidx):
    mesh = plsc.VectorSubcoreMesh(core_axis_name="c", subcore_axis_name="s")
    agg_fn = pl.kernel(
        _agg_body,
        out_type=jax.ShapeDtypeStruct((2, _ET, _N, _DH), jnp.float32),
        mesh=mesh,
        scratch_types=[
            pltpu.VMEM((_NCHUNK, _CHUNK), jnp.int32),
            pltpu.VMEM((_NCHUNK, _CHUNK), jnp.int32),
            pltpu.VMEM((2, _CHUNK, _DH), jnp.float32),
            pltpu.VMEM_SHARED((_ACC_PAD, _DH), jnp.float32),
            pltpu.SemaphoreType.DMA,
            pltpu.SemaphoreType.DMA,
        ],
        compiler_params=pltpu.CompilerParams(use_tc_tiling_on_sc=False),
    )
    return agg_fn(m_lo, m_hi, gidx, sidx)


def _node_mlp_body(x_ref, a00_ref, a01_ref, a10_ref, a11_ref, a20_ref, a21_ref,
                   nt_ref, w1_ref, b1_ref, w2_ref, b2_ref, out_ref):
    x = x_ref[...]
    rx = jnp.maximum(x, 0.0)
    aggs = [(a00_ref[...], a01_ref[...]),
            (a10_ref[...], a11_ref[...]),
            (a20_ref[...], a21_ref[...])]
    outs = []
    for t in range(_NT):
        h = jnp.dot(rx, w1_ref[t, 0], preferred_element_type=jnp.float32) + b1_ref[t]
        for i in range(_ET):
            w = w1_ref[t, i + 1]
            h = h + jnp.dot(aggs[i][0], w[:_DH], preferred_element_type=jnp.float32)
            h = h + jnp.dot(aggs[i][1], w[_DH:], preferred_element_type=jnp.float32)
        h = jnp.maximum(h, 0.0)
        outs.append(jnp.dot(h, w2_ref[t], preferred_element_type=jnp.float32) + b2_ref[t])
    nt = nt_ref[...]
    out_ref[...] = jnp.where(nt == 0, outs[0], outs[1]) + x


def _node_mlp(x, agg, node_type, w1s, b1s, w2s, b2s):
    bn = 2000
    nb = _N // bn
    row_spec = pl.BlockSpec((bn, _DH), lambda j: (j, 0))
    return pl.pallas_call(
        _node_mlp_body,
        grid=(nb,),
        in_specs=[
            pl.BlockSpec((bn, _D), lambda j: (j, 0)),
            row_spec, row_spec, row_spec, row_spec, row_spec, row_spec,
            pl.BlockSpec((bn, 1), lambda j: (j, 0)),
            pl.BlockSpec((_NT, 4, _D, _H), lambda j: (0, 0, 0, 0)),
            pl.BlockSpec((_NT, _H), lambda j: (0, 0)),
            pl.BlockSpec((_NT, _H, _D), lambda j: (0, 0, 0)),
            pl.BlockSpec((_NT, _D), lambda j: (0, 0)),
        ],
        out_specs=pl.BlockSpec((bn, _D), lambda j: (j, 0)),
        out_shape=jax.ShapeDtypeStruct((_N, _D), jnp.float32),
    )(x, agg[0, 0], agg[1, 0], agg[0, 1], agg[1, 1], agg[0, 2], agg[1, 2],
      node_type, w1s, b1s, w2s, b2s)


def kernel(node_feature, edge_index, edge_type, node_type, params):
    ew1 = jnp.stack([p[0] for p in params["edge"]])
    eb1 = jnp.stack([p[1] for p in params["edge"]])[:, None, :]
    ew2 = jnp.stack([p[2] for p in params["edge"]])
    eb2 = jnp.stack([p[3] for p in params["edge"]])[:, None, :]
    nw1 = jnp.stack([p[0] for p in params["node"]]).reshape(_NT, 4, _D, _H)
    nb1 = jnp.stack([p[1] for p in params["node"]])
    nw2 = jnp.stack([p[2] for p in params["node"]])
    nb2 = jnp.stack([p[3] for p in params["node"]])

    m_lo, m_hi = _edge_tables(node_feature, ew1, eb1, ew2, eb2)
    m_lo = m_lo.reshape(_ET * _N, _DH)
    m_hi = m_hi.reshape(_ET * _N, _DH)

    pad = _EPAD - _E
    src = jnp.pad(edge_index[0], (0, pad))
    dst = jnp.pad(edge_index[1], (0, pad), constant_values=-1)
    typ = jnp.pad(edge_type, (0, pad))
    gidx, sidx = _edge_indices(src, dst, typ)

    agg = _aggregate(m_lo, m_hi, gidx, sidx)

    return _node_mlp(node_feature, agg, node_type[:, None], nw1, nb1, nw2, nb2)


# speedup vs baseline: 4.3965x; 1.0194x over previous
import functools

import jax
import jax.numpy as jnp
from jax import lax
from jax.experimental import pallas as pl
from jax.experimental.pallas import tpu as pltpu
from jax.experimental.pallas import tpu_sc as plsc

_N = 10000
_E = 320000
_D = 128
_H = 128
_NT = 2
_ET = 3

_NC = 2
_NS = 16
_L = 16
_DH = _D // 2

_NH = _N // _NC
_ACC_ROWS = _ET * _NH
_TRASH = _ACC_ROWS

_CHUNK = 128
_ACC_PAD = -(-(_ACC_ROWS + 1) // _CHUNK) * _CHUNK

_NBUF = 3
_NCHUNK = _NBUF * (-(-_E // (_NS * _CHUNK * _NBUF)))
_NGROUP = _NCHUNK // _NBUF
_EPT = _NCHUNK * _CHUNK
_EPAD = _EPT * _NS

_ZCHUNKS = _ACC_PAD // _CHUNK
_ZCOPIES = -(-_ZCHUNKS // _NS)
_WB = 320


def _edge_mlp_body(x_ref, w1_ref, b1_ref, w2_ref, b2_ref, lo_ref, hi_ref):
    x = x_ref[...]
    h = jnp.dot(x, w1_ref[0], preferred_element_type=jnp.float32) + b1_ref[0, 0]
    h = jnp.maximum(h, 0.0)
    o = jnp.dot(h, w2_ref[0], preferred_element_type=jnp.float32) + b2_ref[0, 0]
    o = jnp.maximum(o, 0.0)
    lo_ref[0] = o[:, :_DH]
    hi_ref[0] = o[:, _DH:]


def _edge_tables(x, w1s, b1s, w2s, b2s):
    bn = 2000
    nb = _N // bn
    return pl.pallas_call(
        _edge_mlp_body,
        grid=(_ET, nb),
        in_specs=[
            pl.BlockSpec((bn, _D), lambda i, j: (j, 0)),
            pl.BlockSpec((1, _D, _H), lambda i, j: (i, 0, 0)),
            pl.BlockSpec((1, 1, _H), lambda i, j: (i, 0, 0)),
            pl.BlockSpec((1, _H, _D), lambda i, j: (i, 0, 0)),
            pl.BlockSpec((1, 1, _D), lambda i, j: (i, 0, 0)),
        ],
        out_specs=[
            pl.BlockSpec((1, bn, _DH), lambda i, j: (i, j, 0)),
            pl.BlockSpec((1, bn, _DH), lambda i, j: (i, j, 0)),
        ],
        out_shape=[
            jax.ShapeDtypeStruct((_ET, _N, _DH), jnp.float32),
            jax.ShapeDtypeStruct((_ET, _N, _DH), jnp.float32),
        ],
    )(x, w1s, b1s, w2s, b2s)


def _idx_body(src_ref, dst_ref, typ_ref, gidx_ref, sidx_ref):
    src = src_ref[...]
    dst = dst_ref[...]
    typ = typ_ref[...]
    gidx_ref[...] = typ * _N + src
    rel0 = typ * _NH + dst
    trash = _TRASH + jax.lax.broadcasted_iota(jnp.int32, src.shape, 1) % 64
    sidx_ref[0] = jnp.where((dst >= 0) & (dst < _NH), rel0, trash)
    sidx_ref[1] = jnp.where(dst >= _NH, rel0 - _NH, trash)


def _edge_indices(src, dst, typ):
    rows = _EPAD // _CHUNK
    src2 = src.reshape(rows, _CHUNK)
    dst2 = dst.reshape(rows, _CHUNK)
    typ2 = typ.reshape(rows, _CHUNK)
    gidx, sidx = pl.pallas_call(
        _idx_body,
        out_shape=[
            jax.ShapeDtypeStruct((rows, _CHUNK), jnp.int32),
            jax.ShapeDtypeStruct((2, rows, _CHUNK), jnp.int32),
        ],
    )(src2, dst2, typ2)
    return (gidx.reshape(_NS, _NCHUNK, _CHUNK),
            sidx.reshape(2, _NS, _NCHUNK, _CHUNK))


def _agg_body(mlo_hbm, mhi_hbm, gidx_hbm, sidx_hbm, out_hbm,
              gidx_v, sidx_v, rows_v, acc_sh,
              semg0, semg1, semg2, sems0, sems1, sems2):
    cid = lax.axis_index("c")
    sid = lax.axis_index("s")
    base = cid * _NH
    semg = (semg0, semg1, semg2)
    sems = (sems0, sems1, sems2)

    pltpu.sync_copy(gidx_hbm.at[sid], gidx_v)
    pltpu.sync_copy(sidx_hbm.at[cid, sid], sidx_v)

    for p in range(2):
        m_hbm = mlo_hbm if p == 0 else mhi_hbm

        def gstart(c, b, m_hbm=m_hbm):
            pltpu.async_copy(m_hbm.at[gidx_v.at[c]], rows_v.at[b], semg[b])

        def gwait(b, m_hbm=m_hbm):
            pltpu.make_async_copy(m_hbm.at[gidx_v.at[0]], rows_v.at[b],
                                  semg[b]).wait()

        def sstart(c, b):
            pltpu.async_copy(rows_v.at[b], acc_sh.at[sidx_v.at[c]], sems[b],
                             add=True)

        def swait(b):
            pltpu.make_async_copy(rows_v.at[b], acc_sh.at[sidx_v.at[0]],
                                  sems[b]).wait()

        zero = jnp.zeros((_L,), jnp.float32)

        def zrow(j, _):
            for k in range(_DH // _L):
                rows_v[0, j, pl.ds(k * _L, _L)] = zero
            return 0

        lax.fori_loop(0, _CHUNK, zrow, 0)

        def zcopy(j, _):
            idx = jnp.minimum(sid * _ZCOPIES + j, _ZCHUNKS - 1)
            zstart = pl.multiple_of(idx * _CHUNK, _CHUNK)
            pltpu.sync_copy(rows_v.at[0], acc_sh.at[pl.ds(zstart, _CHUNK)])
            return 0

        lax.fori_loop(0, _ZCOPIES, zcopy, 0)
        plsc.subcore_barrier()

        for b in range(_NBUF):
            gstart(b, b)

        def group(g, _):
            c0 = g * _NBUF
            for b in range(_NBUF):
                gwait(b)
                sstart(c0 + b, b)
            for b in range(_NBUF):
                cn = c0 + _NBUF + b

                @pl.when(cn < _NCHUNK)
                def _(b=b, cn=cn):
                    swait(b)
                    gstart(cn, b)

            return 0

        lax.fori_loop(0, _NGROUP, group, 0)
        for b in range(_NBUF):
            swait(b)
        plsc.subcore_barrier()

        wstart = pl.multiple_of(jnp.minimum(sid * _WB, _NH - _WB), 8)
        for i in range(_ET):
            pltpu.sync_copy(acc_sh.at[pl.ds(i * _NH + wstart, _WB)],
                            out_hbm.at[p, i, pl.ds(base + wstart, _WB)])
        plsc.subcore_barrier()


def _aggregate(m_lo, m_hi, gidx, sidx):
    mesh = plsc.VectorSubcoreMesh(core_axis_name="c", subcore_axis_name="s")
    agg_fn = pl.kernel(
        _agg_body,
        out_type=jax.ShapeDtypeStruct((2, _ET, _N, _DH), jnp.float32),
        mesh=mesh,
        scratch_types=[
            pltpu.VMEM((_NCHUNK, _CHUNK), jnp.int32),
            pltpu.VMEM((_NCHUNK, _CHUNK), jnp.int32),
            pltpu.VMEM((_NBUF, _CHUNK, _DH), jnp.float32),
            pltpu.VMEM_SHARED((_ACC_PAD, _DH), jnp.float32),
            pltpu.SemaphoreType.DMA,
            pltpu.SemaphoreType.DMA,
            pltpu.SemaphoreType.DMA,
            pltpu.SemaphoreType.DMA,
            pltpu.SemaphoreType.DMA,
            pltpu.SemaphoreType.DMA,
        ],
        compiler_params=pltpu.CompilerParams(use_tc_tiling_on_sc=False),
    )
    return agg_fn(m_lo, m_hi, gidx, sidx)


def _node_mlp_body(x_ref, a00_ref, a01_ref, a10_ref, a11_ref, a20_ref, a21_ref,
                   nt_ref, w1_ref, b1_ref, w2_ref, b2_ref, out_ref):
    x = x_ref[...]
    rx = jnp.maximum(x, 0.0)
    aggs = [(a00_ref[...], a01_ref[...]),
            (a10_ref[...], a11_ref[...]),
            (a20_ref[...], a21_ref[...])]
    outs = []
    for t in range(_NT):
        h = jnp.dot(rx, w1_ref[t, 0], preferred_element_type=jnp.float32) + b1_ref[t]
        for i in range(_ET):
            w = w1_ref[t, i + 1]
            h = h + jnp.dot(aggs[i][0], w[:_DH], preferred_element_type=jnp.float32)
            h = h + jnp.dot(aggs[i][1], w[_DH:], preferred_element_type=jnp.float32)
        h = jnp.maximum(h, 0.0)
        outs.append(jnp.dot(h, w2_ref[t], preferred_element_type=jnp.float32) + b2_ref[t])
    nt = nt_ref[...]
    out_ref[...] = jnp.where(nt == 0, outs[0], outs[1]) + x


def _node_mlp(x, agg, node_type, w1s, b1s, w2s, b2s):
    bn = 2000
    nb = _N // bn
    row_spec = pl.BlockSpec((bn, _DH), lambda j: (j, 0))
    return pl.pallas_call(
        _node_mlp_body,
        grid=(nb,),
        in_specs=[
            pl.BlockSpec((bn, _D), lambda j: (j, 0)),
            row_spec, row_spec, row_spec, row_spec, row_spec, row_spec,
            pl.BlockSpec((bn, 1), lambda j: (j, 0)),
            pl.BlockSpec((_NT, 4, _D, _H), lambda j: (0, 0, 0, 0)),
            pl.BlockSpec((_NT, _H), lambda j: (0, 0)),
            pl.BlockSpec((_NT, _H, _D), lambda j: (0, 0, 0)),
            pl.BlockSpec((_NT, _D), lambda j: (0, 0)),
        ],
        out_specs=pl.BlockSpec((bn, _D), lambda j: (j, 0)),
        out_shape=jax.ShapeDtypeStruct((_N, _D), jnp.float32),
    )(x, agg[0, 0], agg[1, 0], agg[0, 1], agg[1, 1], agg[0, 2], agg[1, 2],
      node_type, w1s, b1s, w2s, b2s)


def kernel(node_feature, edge_index, edge_type, node_type, params):
    ew1 = jnp.stack([p[0] for p in params["edge"]])
    eb1 = jnp.stack([p[1] for p in params["edge"]])[:, None, :]
    ew2 = jnp.stack([p[2] for p in params["edge"]])
    eb2 = jnp.stack([p[3] for p in params["edge"]])[:, None, :]
    nw1 = jnp.stack([p[0] for p in params["node"]]).reshape(_NT, 4, _D, _H)
    nb1 = jnp.stack([p[1] for p in params["node"]])
    nw2 = jnp.stack([p[2] for p in params["node"]])
    nb2 = jnp.stack([p[3] for p in params["node"]])

    m_lo, m_hi = _edge_tables(node_feature, ew1, eb1, ew2, eb2)
    m_lo = m_lo.reshape(_ET * _N, _DH)
    m_hi = m_hi.reshape(_ET * _N, _DH)

    pad = _EPAD - _E
    src = jnp.pad(edge_index[0], (0, pad))
    dst = jnp.pad(edge_index[1], (0, pad), constant_values=-1)
    typ = jnp.pad(edge_type, (0, pad))
    gidx, sidx = _edge_indices(src, dst, typ)

    agg = _aggregate(m_lo, m_hi, gidx, sidx)

    return _node_mlp(node_feature, agg, node_type[:, None], nw1, nb1, nw2, nb2)


# speedup vs baseline: 4.5545x; 1.0359x over previous
import jax
import jax.numpy as jnp
from jax import lax
from jax.experimental import pallas as pl
from jax.experimental.pallas import tpu as pltpu
from jax.experimental.pallas import tpu_sc as plsc

_N = 10000
_E = 320000
_D = 128
_H = 128
_NT = 2
_ET = 3

_NC = 2
_NS = 16
_L = 16
_NP = 4
_DQ = _D // _NP

_ACC_ROWS = _ET * _N
_TRASH = _ACC_ROWS

_CHUNK = 128
_ACC_PAD = -(-(_ACC_ROWS + 64) // _CHUNK) * _CHUNK

_NBUF = 4
_NCHUNK = _NBUF * (-(-_E // (_NC * _NS * _CHUNK * _NBUF)))
_NGRP = _NCHUNK // _NBUF
_EPT = _NCHUNK * _CHUNK
_EPAD = _EPT * _NC * _NS

_ZCHUNKS = _ACC_PAD // _CHUNK
_ZCOPIES = -(-_ZCHUNKS // _NS)
_WB = 632


def _edge_mlp_body(x_ref, w1_ref, b1_ref, w2_ref, b2_ref, *out_refs):
    x = x_ref[...]
    h = jnp.dot(x, w1_ref[0], preferred_element_type=jnp.float32) + b1_ref[0, 0]
    h = jnp.maximum(h, 0.0)
    o = jnp.dot(h, w2_ref[0], preferred_element_type=jnp.float32) + b2_ref[0, 0]
    o = jnp.maximum(o, 0.0)
    for p in range(_NP):
        out_refs[p][0] = o[:, p * _DQ:(p + 1) * _DQ]


def _edge_tables(x, w1s, b1s, w2s, b2s):
    bn = 2000
    nb = _N // bn
    return pl.pallas_call(
        _edge_mlp_body,
        grid=(_ET, nb),
        in_specs=[
            pl.BlockSpec((bn, _D), lambda i, j: (j, 0)),
            pl.BlockSpec((1, _D, _H), lambda i, j: (i, 0, 0)),
            pl.BlockSpec((1, 1, _H), lambda i, j: (i, 0, 0)),
            pl.BlockSpec((1, _H, _D), lambda i, j: (i, 0, 0)),
            pl.BlockSpec((1, 1, _D), lambda i, j: (i, 0, 0)),
        ],
        out_specs=[pl.BlockSpec((1, bn, _DQ), lambda i, j: (i, j, 0))
                   for _ in range(_NP)],
        out_shape=[jax.ShapeDtypeStruct((_ET, _N, _DQ), jnp.float32)
                   for _ in range(_NP)],
    )(x, w1s, b1s, w2s, b2s)


def _idx_body(src_ref, dst_ref, typ_ref, gidx_ref, sidx_ref):
    src = src_ref[...]
    dst = dst_ref[...]
    typ = typ_ref[...]
    gidx_ref[...] = typ * _N + src
    trash = _TRASH + jax.lax.broadcasted_iota(jnp.int32, src.shape, 1) % 64
    sidx_ref[...] = jnp.where(dst >= 0, typ * _N + dst, trash)


def _edge_indices(src, dst, typ):
    rows = _EPAD // _CHUNK
    src2 = src.reshape(rows, _CHUNK)
    dst2 = dst.reshape(rows, _CHUNK)
    typ2 = typ.reshape(rows, _CHUNK)
    gidx, sidx = pl.pallas_call(
        _idx_body,
        out_shape=[
            jax.ShapeDtypeStruct((rows, _CHUNK), jnp.int32),
            jax.ShapeDtypeStruct((rows, _CHUNK), jnp.int32),
        ],
    )(src2, dst2, typ2)
    return (gidx.reshape(_NC, _NS, _NCHUNK, _CHUNK),
            sidx.reshape(_NC, _NS, _NCHUNK, _CHUNK))


def _agg_body(m0_hbm, m1_hbm, m2_hbm, m3_hbm, gidx_hbm, sidx_hbm, out_hbm,
              gidx_v, sidx_v, rows_v, acc_sh,
              semg0, semg1, semg2, semg3, sems0, sems1, sems2, sems3):
    cid = lax.axis_index("c")
    sid = lax.axis_index("s")
    semg = (semg0, semg1, semg2, semg3)
    sems = (sems0, sems1, sems2, sems3)

    pltpu.sync_copy(gidx_hbm.at[cid, sid], gidx_v)
    pltpu.sync_copy(sidx_hbm.at[cid, sid], sidx_v)

    for p, m_hbm in enumerate((m0_hbm, m1_hbm, m2_hbm, m3_hbm)):

        def gstart(c, b, m_hbm=m_hbm):
            pltpu.async_copy(m_hbm.at[gidx_v.at[c]], rows_v.at[b], semg[b])

        def gwait(b, m_hbm=m_hbm):
            pltpu.make_async_copy(m_hbm.at[gidx_v.at[0]], rows_v.at[b],
                                  semg[b]).wait()

        def sstart(c, b):
            pltpu.async_copy(rows_v.at[b], acc_sh.at[sidx_v.at[c]], sems[b],
                             add=True)

        def swait(b):
            pltpu.make_async_copy(rows_v.at[b], acc_sh.at[sidx_v.at[0]],
                                  sems[b]).wait()

        zero = jnp.zeros((_L,), jnp.float32)

        def zrow(j, _):
            for k in range(_DQ // _L):
                rows_v[0, j, pl.ds(k * _L, _L)] = zero
            return 0

        lax.fori_loop(0, _CHUNK, zrow, 0)

        def zcopy(j, _):
            idx = jnp.minimum(sid * _ZCOPIES + j, _ZCHUNKS - 1)
            zstart = pl.multiple_of(idx * _CHUNK, _CHUNK)
            pltpu.sync_copy(rows_v.at[0], acc_sh.at[pl.ds(zstart, _CHUNK)])
            return 0

        lax.fori_loop(0, _ZCOPIES, zcopy, 0)
        plsc.subcore_barrier()

        for b in range(_NBUF):
            gstart(b, b)

        def group(g, _):
            c0 = g * _NBUF
            for b in range(_NBUF):
                gwait(b)
                sstart(c0 + b, b)
            for b in range(_NBUF):
                @pl.when(g + 1 < _NGRP)
                def _(b=b):
                    swait(b)
                    gstart((g + 1) * _NBUF + b, b)

            return 0

        lax.fori_loop(0, _NGRP, group, 0)
        for b in range(_NBUF):
            swait(b)
        plsc.subcore_barrier()

        wstart = pl.multiple_of(jnp.minimum(sid * _WB, _N - _WB), 8)
        for i in range(_ET):
            pltpu.sync_copy(acc_sh.at[pl.ds(i * _N + wstart, _WB)],
                            out_hbm.at[cid, p, i, pl.ds(wstart, _WB)])
        plsc.subcore_barrier()


def _aggregate(ms, gidx, sidx):
    mesh = plsc.VectorSubcoreMesh(core_axis_name="c", subcore_axis_name="s")
    agg_fn = pl.kernel(
        _agg_body,
        out_type=jax.ShapeDtypeStruct((_NC, _NP, _ET, _N, _DQ), jnp.float32),
        mesh=mesh,
        scratch_types=[
            pltpu.VMEM((_NCHUNK, _CHUNK), jnp.int32),
            pltpu.VMEM((_NCHUNK, _CHUNK), jnp.int32),
            pltpu.VMEM((_NBUF, _CHUNK, _DQ), jnp.float32),
            pltpu.VMEM_SHARED((_ACC_PAD, _DQ), jnp.float32),
            pltpu.SemaphoreType.DMA,
            pltpu.SemaphoreType.DMA,
            pltpu.SemaphoreType.DMA,
            pltpu.SemaphoreType.DMA,
            pltpu.SemaphoreType.DMA,
            pltpu.SemaphoreType.DMA,
            pltpu.SemaphoreType.DMA,
            pltpu.SemaphoreType.DMA,
        ],
        compiler_params=pltpu.CompilerParams(use_tc_tiling_on_sc=False),
    )
    return agg_fn(*ms, gidx, sidx)


def _node_mlp_body(x_ref, agg_ref, nt_ref, w1_ref, b1_ref, w2_ref, b2_ref,
                   out_ref):
    x = x_ref[...]
    rx = jnp.maximum(x, 0.0)
    a = agg_ref[...]
    outs = []
    for t in range(_NT):
        h = jnp.dot(rx, w1_ref[t, 0], preferred_element_type=jnp.float32) + b1_ref[t]
        for i in range(_ET):
            w = w1_ref[t, i + 1]
            for p in range(_NP):
                ap = a[0, p, i] + a[1, p, i]
                h = h + jnp.dot(ap, w[p * _DQ:(p + 1) * _DQ],
                                preferred_element_type=jnp.float32)
        h = jnp.maximum(h, 0.0)
        outs.append(jnp.dot(h, w2_ref[t], preferred_element_type=jnp.float32) + b2_ref[t])
    nt = nt_ref[...]
    out_ref[...] = jnp.where(nt == 0, outs[0], outs[1]) + x


def _node_mlp(x, agg, node_type, w1s, b1s, w2s, b2s):
    bn = 1000
    nb = _N // bn
    return pl.pallas_call(
        _node_mlp_body,
        grid=(nb,),
        in_specs=[
            pl.BlockSpec((bn, _D), lambda j: (j, 0)),
            pl.BlockSpec((_NC, _NP, _ET, bn, _DQ), lambda j: (0, 0, 0, j, 0)),
            pl.BlockSpec((bn, 1), lambda j: (j, 0)),
            pl.BlockSpec((_NT, 4, _D, _H), lambda j: (0, 0, 0, 0)),
            pl.BlockSpec((_NT, _H), lambda j: (0, 0)),
            pl.BlockSpec((_NT, _H, _D), lambda j: (0, 0, 0)),
            pl.BlockSpec((_NT, _D), lambda j: (0, 0)),
        ],
        out_specs=pl.BlockSpec((bn, _D), lambda j: (j, 0)),
        out_shape=jax.ShapeDtypeStruct((_N, _D), jnp.float32),
    )(x, agg, node_type, w1s, b1s, w2s, b2s)


def kernel(node_feature, edge_index, edge_type, node_type, params):
    ew1 = jnp.stack([p[0] for p in params["edge"]])
    eb1 = jnp.stack([p[1] for p in params["edge"]])[:, None, :]
    ew2 = jnp.stack([p[2] for p in params["edge"]])
    eb2 = jnp.stack([p[3] for p in params["edge"]])[:, None, :]
    nw1 = jnp.stack([p[0] for p in params["node"]]).reshape(_NT, 4, _D, _H)
    nb1 = jnp.stack([p[1] for p in params["node"]])
    nw2 = jnp.stack([p[2] for p in params["node"]])
    nb2 = jnp.stack([p[3] for p in params["node"]])

    ms = _edge_tables(node_feature, ew1, eb1, ew2, eb2)
    ms = [m.reshape(_ET * _N, _DQ) for m in ms]

    pad = _EPAD - _E
    src = jnp.pad(edge_index[0], (0, pad))
    dst = jnp.pad(edge_index[1], (0, pad), constant_values=-1)
    typ = jnp.pad(edge_type, (0, pad))
    gidx, sidx = _edge_indices(src, dst, typ)

    agg = _aggregate(ms, gidx, sidx)

    return _node_mlp(node_feature, agg, node_type[:, None], nw1, nb1, nw2, nb2)


# speedup vs baseline: 4.6136x; 1.0130x over previous
import jax
import jax.numpy as jnp
from jax import lax
from jax.experimental import pallas as pl
from jax.experimental.pallas import tpu as pltpu
from jax.experimental.pallas import tpu_sc as plsc

_N = 10000
_E = 320000
_D = 128
_H = 128
_NT = 2
_ET = 3

_NC = 2
_NS = 16
_L = 16
_NP = 4
_DQ = _D // _NP

_ACC_ROWS = _ET * _N
_TRASH = _ACC_ROWS

_CHUNK = 128
_ACC_PAD = -(-(_ACC_ROWS + 64) // _CHUNK) * _CHUNK

_NBUF = 8
_NCHUNK = _NBUF * (-(-_E // (_NC * _NS * _CHUNK * _NBUF)))
_NGRP = _NCHUNK // _NBUF
_EPT = _NCHUNK * _CHUNK
_EPAD = _EPT * _NC * _NS

_ZCHUNKS = _ACC_PAD // _CHUNK
_ZCOPIES = -(-_ZCHUNKS // _NS)
_WB = 632


def _edge_mlp_body(x_ref, w1_ref, b1_ref, w2_ref, b2_ref, *out_refs):
    x = x_ref[...]
    h = jnp.dot(x, w1_ref[0], preferred_element_type=jnp.float32) + b1_ref[0, 0]
    h = jnp.maximum(h, 0.0)
    o = jnp.dot(h, w2_ref[0], preferred_element_type=jnp.float32) + b2_ref[0, 0]
    o = jnp.maximum(o, 0.0)
    for p in range(_NP):
        out_refs[p][0] = o[:, p * _DQ:(p + 1) * _DQ]


def _edge_tables(x, w1s, b1s, w2s, b2s):
    bn = 2000
    nb = _N // bn
    return pl.pallas_call(
        _edge_mlp_body,
        grid=(_ET, nb),
        in_specs=[
            pl.BlockSpec((bn, _D), lambda i, j: (j, 0)),
            pl.BlockSpec((1, _D, _H), lambda i, j: (i, 0, 0)),
            pl.BlockSpec((1, 1, _H), lambda i, j: (i, 0, 0)),
            pl.BlockSpec((1, _H, _D), lambda i, j: (i, 0, 0)),
            pl.BlockSpec((1, 1, _D), lambda i, j: (i, 0, 0)),
        ],
        out_specs=[pl.BlockSpec((1, bn, _DQ), lambda i, j: (i, j, 0))
                   for _ in range(_NP)],
        out_shape=[jax.ShapeDtypeStruct((_ET, _N, _DQ), jnp.float32)
                   for _ in range(_NP)],
    )(x, w1s, b1s, w2s, b2s)


def _idx_body(src_ref, dst_ref, typ_ref, gidx_ref, sidx_ref):
    src = src_ref[...]
    dst = dst_ref[...]
    typ = typ_ref[...]
    gidx_ref[...] = typ * _N + src
    trash = _TRASH + jax.lax.broadcasted_iota(jnp.int32, src.shape, 1) % 64
    sidx_ref[...] = jnp.where(dst >= 0, typ * _N + dst, trash)


def _edge_indices(src, dst, typ):
    rows = _EPAD // _CHUNK
    src2 = src.reshape(rows, _CHUNK)
    dst2 = dst.reshape(rows, _CHUNK)
    typ2 = typ.reshape(rows, _CHUNK)
    gidx, sidx = pl.pallas_call(
        _idx_body,
        out_shape=[
            jax.ShapeDtypeStruct((rows, _CHUNK), jnp.int32),
            jax.ShapeDtypeStruct((rows, _CHUNK), jnp.int32),
        ],
    )(src2, dst2, typ2)
    return (gidx.reshape(_NC, _NS, _NCHUNK, _CHUNK),
            sidx.reshape(_NC, _NS, _NCHUNK, _CHUNK))


def _agg_body(m0_hbm, m1_hbm, m2_hbm, m3_hbm, gidx_hbm, sidx_hbm, out_hbm,
              gidx_v, sidx_v, rows_v, acc_sh, *sems_all):
    cid = lax.axis_index("c")
    sid = lax.axis_index("s")
    semg = sems_all[:_NBUF]
    sems = sems_all[_NBUF:]

    pltpu.sync_copy(gidx_hbm.at[cid, sid], gidx_v)
    pltpu.sync_copy(sidx_hbm.at[cid, sid], sidx_v)

    for p, m_hbm in enumerate((m0_hbm, m1_hbm, m2_hbm, m3_hbm)):

        def gstart(c, b, m_hbm=m_hbm):
            pltpu.async_copy(m_hbm.at[gidx_v.at[c]], rows_v.at[b], semg[b])

        def gwait(b, m_hbm=m_hbm):
            pltpu.make_async_copy(m_hbm.at[gidx_v.at[0]], rows_v.at[b],
                                  semg[b]).wait()

        def sstart(c, b):
            pltpu.async_copy(rows_v.at[b], acc_sh.at[sidx_v.at[c]], sems[b],
                             add=True)

        def swait(b):
            pltpu.make_async_copy(rows_v.at[b], acc_sh.at[sidx_v.at[0]],
                                  sems[b]).wait()

        zero = jnp.zeros((_L,), jnp.float32)

        def zrow(j, _):
            for k in range(_DQ // _L):
                rows_v[0, j, pl.ds(k * _L, _L)] = zero
            return 0

        lax.fori_loop(0, _CHUNK, zrow, 0)

        def zcopy(j, _):
            idx = jnp.minimum(sid * _ZCOPIES + j, _ZCHUNKS - 1)
            zstart = pl.multiple_of(idx * _CHUNK, _CHUNK)
            pltpu.sync_copy(rows_v.at[0], acc_sh.at[pl.ds(zstart, _CHUNK)])
            return 0

        lax.fori_loop(0, _ZCOPIES, zcopy, 0)
        plsc.subcore_barrier()

        for b in range(_NBUF):
            gstart(b, b)

        def group(g, _):
            c0 = g * _NBUF
            for b in range(_NBUF):
                gwait(b)
                sstart(c0 + b, b)
            for b in range(_NBUF):
                @pl.when(g + 1 < _NGRP)
                def _(b=b):
                    swait(b)
                    gstart((g + 1) * _NBUF + b, b)

            return 0

        lax.fori_loop(0, _NGRP, group, 0)
        for b in range(_NBUF):
            swait(b)
        plsc.subcore_barrier()

        wstart = pl.multiple_of(jnp.minimum(sid * _WB, _N - _WB), 8)
        for i in range(_ET):
            pltpu.sync_copy(acc_sh.at[pl.ds(i * _N + wstart, _WB)],
                            out_hbm.at[cid, p, i, pl.ds(wstart, _WB)])
        plsc.subcore_barrier()


def _aggregate(ms, gidx, sidx):
    mesh = plsc.VectorSubcoreMesh(core_axis_name="c", subcore_axis_name="s")
    agg_fn = pl.kernel(
        _agg_body,
        out_type=jax.ShapeDtypeStruct((_NC, _NP, _ET, _N, _DQ), jnp.float32),
        mesh=mesh,
        scratch_types=[
            pltpu.VMEM((_NCHUNK, _CHUNK), jnp.int32),
            pltpu.VMEM((_NCHUNK, _CHUNK), jnp.int32),
            pltpu.VMEM((_NBUF, _CHUNK, _DQ), jnp.float32),
            pltpu.VMEM_SHARED((_ACC_PAD, _DQ), jnp.float32),
        ] + [pltpu.SemaphoreType.DMA] * (2 * _NBUF),
        compiler_params=pltpu.CompilerParams(use_tc_tiling_on_sc=False),
    )
    return agg_fn(*ms, gidx, sidx)


def _node_mlp_body(x_ref, agg_ref, nt_ref, w1_ref, b1_ref, w2_ref, b2_ref,
                   out_ref):
    x = x_ref[...]
    rx = jnp.maximum(x, 0.0)
    a = agg_ref[...]
    outs = []
    for t in range(_NT):
        h = jnp.dot(rx, w1_ref[t, 0], preferred_element_type=jnp.float32) + b1_ref[t]
        for i in range(_ET):
            w = w1_ref[t, i + 1]
            for p in range(_NP):
                ap = a[0, p, i] + a[1, p, i]
                h = h + jnp.dot(ap, w[p * _DQ:(p + 1) * _DQ],
                                preferred_element_type=jnp.float32)
        h = jnp.maximum(h, 0.0)
        outs.append(jnp.dot(h, w2_ref[t], preferred_element_type=jnp.float32) + b2_ref[t])
    nt = nt_ref[...]
    out_ref[...] = jnp.where(nt == 0, outs[0], outs[1]) + x


def _node_mlp(x, agg, node_type, w1s, b1s, w2s, b2s):
    bn = 1000
    nb = _N // bn
    return pl.pallas_call(
        _node_mlp_body,
        grid=(nb,),
        in_specs=[
            pl.BlockSpec((bn, _D), lambda j: (j, 0)),
            pl.BlockSpec((_NC, _NP, _ET, bn, _DQ), lambda j: (0, 0, 0, j, 0)),
            pl.BlockSpec((bn, 1), lambda j: (j, 0)),
            pl.BlockSpec((_NT, 4, _D, _H), lambda j: (0, 0, 0, 0)),
            pl.BlockSpec((_NT, _H), lambda j: (0, 0)),
            pl.BlockSpec((_NT, _H, _D), lambda j: (0, 0, 0)),
            pl.BlockSpec((_NT, _D), lambda j: (0, 0)),
        ],
        out_specs=pl.BlockSpec((bn, _D), lambda j: (j, 0)),
        out_shape=jax.ShapeDtypeStruct((_N, _D), jnp.float32),
    )(x, agg, node_type, w1s, b1s, w2s, b2s)


def kernel(node_feature, edge_index, edge_type, node_type, params):
    ew1 = jnp.stack([p[0] for p in params["edge"]])
    eb1 = jnp.stack([p[1] for p in params["edge"]])[:, None, :]
    ew2 = jnp.stack([p[2] for p in params["edge"]])
    eb2 = jnp.stack([p[3] for p in params["edge"]])[:, None, :]
    nw1 = jnp.stack([p[0] for p in params["node"]]).reshape(_NT, 4, _D, _H)
    nb1 = jnp.stack([p[1] for p in params["node"]])
    nw2 = jnp.stack([p[2] for p in params["node"]])
    nb2 = jnp.stack([p[3] for p in params["node"]])

    ms = _edge_tables(node_feature, ew1, eb1, ew2, eb2)
    ms = [m.reshape(_ET * _N, _DQ) for m in ms]

    pad = _EPAD - _E
    src = jnp.pad(edge_index[0], (0, pad))
    dst = jnp.pad(edge_index[1], (0, pad), constant_values=-1)
    typ = jnp.pad(edge_type, (0, pad))
    gidx, sidx = _edge_indices(src, dst, typ)

    agg = _aggregate(ms, gidx, sidx)

    return _node_mlp(node_feature, agg, node_type[:, None], nw1, nb1, nw2, nb2)
